# baseline (speedup 1.0000x reference)
import functools

import jax
import jax.numpy as jnp
from jax import lax
from jax.experimental import pallas as pl
from jax.experimental.pallas import tpu as pltpu
from jax.experimental.pallas import tpu_sc as plsc


_NC = 2
_NS = 16
_NW = _NC * _NS
_LANES = 16
_RB = 8


def _sc_body(x_hbm, pos_hbm, out_hbm, *refs):
    b, s, d = x_hbm.shape
    chunk = s // _NW
    nblk = chunk // _RB

    xb = [list(refs[0:4]), list(refs[4:8])]
    ob = [list(refs[8:12]), list(refs[12:16])]
    pb = [refs[16], refs[17]]
    sx = [refs[18], refs[19]]
    so = [refs[20], refs[21]]
    sp = [refs[22], refs[23]]

    wid = lax.axis_index("s") * _NC + lax.axis_index("c")
    base = wid * chunk

    def start_fetch(j, p):
        r0 = base + j * _RB
        pltpu.async_copy(pos_hbm.at[pl.ds(r0, _RB)], pb[p], sp[p])
        for bi in range(b):
            pltpu.async_copy(x_hbm.at[bi, pl.ds(r0, _RB)], xb[p][bi], sx[p])

    start_fetch(0, 0)
    start_fetch(1, 1)

    def pair_loop(g, carry):
        for p in range(2):
            j = 2 * g + p
            r0 = base + j * _RB
            pltpu.make_async_copy(pos_hbm.at[pl.ds(0, _RB)], pb[p], sp[p]).wait()
            for bi in range(b):
                pltpu.make_async_copy(
                    x_hbm.at[bi, pl.ds(0, _RB)], xb[p][bi], sx[p]
                ).wait()

            @pl.when(g > 0)
            def _drain():
                for bi in range(b):
                    pltpu.make_async_copy(
                        ob[p][bi], out_hbm.at[bi, pl.ds(0, _RB)], so[p]
                    ).wait()

            def row_loop(i, c):
                for k in range(d // _LANES):
                    sl = (i, pl.ds(k * _LANES, _LANES))
                    pv = pb[p][sl]
                    for bi in range(b):
                        ob[p][bi][sl] = xb[p][bi][sl] + pv
                return c

            lax.fori_loop(0, _RB, row_loop, 0)

            for bi in range(b):
                pltpu.async_copy(ob[p][bi], out_hbm.at[bi, pl.ds(r0, _RB)], so[p])

            @pl.when(j + 2 < nblk)
            def _prefetch():
                start_fetch(j + 2, p)

        return carry

    lax.fori_loop(0, nblk // 2, pair_loop, 0)

    for p in range(2):
        for bi in range(b):
            pltpu.make_async_copy(
                ob[p][bi], out_hbm.at[bi, pl.ds(0, _RB)], so[p]
            ).wait()


def kernel(x, pos_embedding):
    b, s, d = x.shape

    mesh = plsc.VectorSubcoreMesh(core_axis_name="c", subcore_axis_name="s")
    buf = pltpu.VMEM((_RB, d), jnp.float32)
    run = functools.partial(
        pl.kernel,
        mesh=mesh,
        out_type=jax.ShapeDtypeStruct((b, s, d), jnp.float32),
        scratch_types=(
            [buf] * 16
            + [buf] * 2
            + [pltpu.SemaphoreType.DMA] * 6
        ),
    )(_sc_body)
    return run(x, pos_embedding)
